# baseline (speedup 1.0000x reference)
import jax
import jax.numpy as jnp
from jax.experimental import pallas as pl
from jax.experimental.pallas import tpu as pltpu
from jax.experimental.pallas import tpu_sc as plsc

VOCAB = 50
POS_DIM = 64
WINDOW = 128
SPREAD = 4
ROW = 2 * POS_DIM


def _sc_gather(pair_table, idx_flat, n_pairs):
    mesh = plsc.VectorSubcoreMesh(core_axis_name="core", subcore_axis_name="subcore")

    @pl.kernel(
        out_type=jax.ShapeDtypeStruct((n_pairs, ROW), pair_table.dtype),
        mesh=mesh,
        scratch_types=[pltpu.SemaphoreType.DMA, pltpu.SemaphoreType.DMA],
    )
    def gather_kernel(table_hbm, idx_hbm, out_hbm, sem_a, sem_b):
        half = WINDOW // 2

        def body(idx_vmem, out_vmem):
            h1 = pltpu.async_copy(
                table_hbm.at[idx_vmem.at[0, pl.ds(0, half)]],
                out_vmem.at[pl.ds(0, half)],
                sem_a,
            )
            h2 = pltpu.async_copy(
                table_hbm.at[idx_vmem.at[0, pl.ds(half, half)]],
                out_vmem.at[pl.ds(half, half)],
                sem_b,
            )
            h1.wait()
            h2.wait()

        pltpu.emit_pipeline(
            body,
            grid=(n_pairs // WINDOW,),
            in_specs=[pl.BlockSpec((1, WINDOW), index_map=lambda i: (0, i))],
            out_specs=[pl.BlockSpec((WINDOW, ROW), index_map=lambda i: (i, 0))],
            core_axis_name=("core", "subcore"),
            dimension_semantics=(pltpu.PARALLEL,),
        )(idx_hbm, out_hbm)

    return gather_kernel(pair_table, idx_flat)


def kernel(indices, table):
    batch, seq_len = indices.shape
    n_pairs = batch * seq_len // 2

    pair_table = jnp.concatenate(
        [
            jnp.broadcast_to(table[:, None, :], (VOCAB, VOCAB, POS_DIM)),
            jnp.broadcast_to(table[None, :, :], (VOCAB, VOCAB, POS_DIM)),
        ],
        axis=-1,
    ).reshape(VOCAB * VOCAB, ROW)
    pair_table = jnp.tile(pair_table, (SPREAD, 1))

    idx = indices.astype(jnp.int32)
    pair_idx = (idx[:, 0::2] * VOCAB + idx[:, 1::2]).reshape(1, n_pairs)
    offs = (jax.lax.iota(jnp.int32, n_pairs) & (SPREAD - 1)).reshape(1, n_pairs)
    pair_idx = pair_idx + (VOCAB * VOCAB) * offs

    wide = _sc_gather(pair_table, pair_idx, n_pairs)
    flat = jax.lax.optimization_barrier(wide.reshape(n_pairs * ROW))
    return flat.reshape(indices.shape[0], seq_len, POS_DIM)


# speedup vs baseline: 1.3484x; 1.3484x over previous
import jax
import jax.numpy as jnp
from jax.experimental import pallas as pl
from jax.experimental.pallas import tpu as pltpu
from jax.experimental.pallas import tpu_sc as plsc

VOCAB = 50
POS_DIM = 64
WINDOW = 128


def _sc_gather(slab_table, idx_flat, n_pairs):
    mesh = plsc.VectorSubcoreMesh(core_axis_name="core", subcore_axis_name="subcore")

    @pl.kernel(
        out_type=jax.ShapeDtypeStruct((n_pairs, 2, 2 * POS_DIM), slab_table.dtype),
        mesh=mesh,
        scratch_types=[pltpu.SemaphoreType.DMA, pltpu.SemaphoreType.DMA],
    )
    def gather_kernel(table_hbm, idx_hbm, out_hbm, sem_a, sem_b):
        half = WINDOW // 2

        def body(idx_vmem, out_vmem):
            h1 = pltpu.async_copy(
                table_hbm.at[idx_vmem.at[0, pl.ds(0, half)]],
                out_vmem.at[pl.ds(0, half)],
                sem_a,
            )
            h2 = pltpu.async_copy(
                table_hbm.at[idx_vmem.at[0, pl.ds(half, half)]],
                out_vmem.at[pl.ds(half, half)],
                sem_b,
            )
            h1.wait()
            h2.wait()

        pltpu.emit_pipeline(
            body,
            grid=(n_pairs // WINDOW,),
            in_specs=[pl.BlockSpec((1, WINDOW), index_map=lambda i: (0, i))],
            out_specs=[
                pl.BlockSpec(
                    (WINDOW, 2, 2 * POS_DIM), index_map=lambda i: (i, 0, 0)
                )
            ],
            core_axis_name=("core", "subcore"),
            dimension_semantics=(pltpu.PARALLEL,),
        )(idx_hbm, out_hbm)

    return gather_kernel(slab_table, idx_flat)


def kernel(indices, table):
    batch, seq_len = indices.shape
    n_pairs = batch * seq_len // 2

    rep = jnp.concatenate([table, table], axis=1)
    slab_table = jnp.stack(
        [
            jnp.broadcast_to(rep[:, None, :], (VOCAB, VOCAB, 2 * POS_DIM)),
            jnp.broadcast_to(rep[None, :, :], (VOCAB, VOCAB, 2 * POS_DIM)),
        ],
        axis=2,
    ).reshape(VOCAB * VOCAB, 2, 2 * POS_DIM)

    slab_table = jnp.tile(slab_table, (8, 1, 1))
    idx = indices.astype(jnp.int32)
    pair_idx = (idx[:, 0::2] * VOCAB + idx[:, 1::2]).reshape(1, n_pairs)
    offs = (jax.lax.iota(jnp.int32, n_pairs) & 7).reshape(1, n_pairs)
    pair_idx = pair_idx + (VOCAB * VOCAB) * offs

    wide = _sc_gather(slab_table, pair_idx, n_pairs)
    return wide.reshape(batch, seq_len, 2 * POS_DIM)[:, :, :POS_DIM]
